# baseline (speedup 1.0000x reference)
import numpy as np
import jax
import jax.numpy as jnp
from jax.experimental import pallas as pl
from jax.experimental.pallas import tpu as pltpu

N_NEG = 24
TEMP = 0.07
B, D, H, W = 16, 32, 224, 224
TB = 2 * B
HW = H * W

_perms = np.stack([np.random.default_rng(1000 + i).permutation(TB)[:N_NEG]
                   for i in range(TB)])
_sel = np.zeros((TB, TB), np.float32)
for _i in range(TB):
    _sel[_i, _perms[_i]] = 1.0


KH = 4
HC = H // KH


def _stage1_kernel(feat_ref, lab_ref, out_ref):
    k = pl.program_id(1)

    @pl.when(k == 0)
    def _init():
        out_ref[0, :, 0:8] = jnp.zeros((D, 8), jnp.float32)

    f = feat_ref[0]
    l0 = lab_ref[0, 0]
    l1 = lab_ref[0, 1]
    st0 = jnp.sum(f * l0[None, :, :], axis=(1, 2), keepdims=True)
    st1 = jnp.sum(f * l1[None, :, :], axis=(1, 2), keepdims=True)
    tot = jnp.sum(f, axis=(1, 2), keepdims=True)
    out_ref[0, :, 0:1] += st0.reshape(D, 1)
    out_ref[0, :, 1:2] += st1.reshape(D, 1)
    out_ref[0, :, 2:3] += tot.reshape(D, 1)
    cnt0 = jnp.sum(l0, axis=(0, 1), keepdims=True)
    cnt1 = jnp.sum(l1, axis=(0, 1), keepdims=True)
    out_ref[0, 0:1, 3:4] += cnt0
    out_ref[0, 0:1, 4:5] += cnt1


def _stage2_kernel(sums_ref, tidc_ref, tidr_ref, sel_ref, out_ref):
    sel = sel_ref[...]

    st_t = jnp.concatenate([sums_ref[b, :, 0:2] for b in range(B)], axis=1)
    tot_t = jnp.concatenate(
        [sums_ref[b, :, 2:3] for b in range(B) for _ in range(2)], axis=1)
    cnt_r = jnp.concatenate([sums_ref[b, 0:1, 3:5] for b in range(B)], axis=1)

    def normalize(v, c):
        v = v / jnp.maximum(c, 1.0)
        n = jnp.sqrt(jnp.sum(v * v, axis=0, keepdims=True))
        return v / jnp.maximum(n, 1e-12)

    tgt = normalize(st_t, cnt_r)
    bgd = normalize(tot_t - st_t, float(HW) - cnt_r)

    dn = (((0,), (0,)), ((), ()))
    lt = jax.lax.dot_general(tgt, tgt, dn,
                             preferred_element_type=jnp.float32) / TEMP
    lb = jax.lax.dot_general(tgt, bgd, dn,
                             preferred_element_type=jnp.float32) / TEMP

    same = tidc_ref[...] == tidr_ref[...]
    df = jnp.where(same, 0.0, 1.0)
    sm = 1.0 - df

    rr = jax.lax.broadcasted_iota(jnp.int32, (TB, TB), 0)
    cc = jax.lax.broadcasted_iota(jnp.int32, (TB, TB), 1)
    upper = jnp.where(rr < cc, 1.0, 0.0)
    dn2 = (((1,), (0,)), ((), ()))
    rd = jax.lax.dot_general(df, upper, dn2, preferred_element_type=jnp.float32)
    rs = jax.lax.dot_general(sm, upper, dn2, preferred_element_type=jnp.float32)
    n_diff = jnp.sum(df, axis=1, keepdims=True)
    posn = jnp.where(same, n_diff + rs, rd)

    ins = jnp.zeros((TB, TB), jnp.float32)
    for p in range(TB):
        ins = ins + sel[:, p:p + 1] * jnp.where(posn == float(p), 1.0, 0.0)

    val = jnp.where(same, lb, lt)
    vmask = jnp.where(ins > 0.5, val, -1e30)
    nmax = jnp.max(vmask, axis=1, keepdims=True)
    sumexp = jnp.sum(ins * jnp.exp(val - nmax), axis=1, keepdims=True)

    cond = same & (cc != tidc_ref[...])
    firstc = jnp.min(jnp.where(cond, cc, TB * 2), axis=1, keepdims=True)
    firstc = jnp.where(firstc == TB * 2, 0, firstc)
    pos_logit = jnp.sum(lt * jnp.where(cc == firstc, 1.0, 0.0),
                        axis=1, keepdims=True)

    loss_i = jnp.log(sumexp) - (pos_logit - nmax)
    out_ref[...] = jnp.sum(loss_i, axis=0, keepdims=True) / TB


def kernel(features, labels, tasks):
    b = features.shape[0]

    sums = pl.pallas_call(
        _stage1_kernel,
        grid=(b, KH),
        in_specs=[
            pl.BlockSpec((1, D, HC, W), lambda i, k: (i, 0, k, 0)),
            pl.BlockSpec((1, 2, HC, W), lambda i, k: (i, 0, k, 0)),
        ],
        out_specs=pl.BlockSpec((1, D, 128), lambda i, k: (i, 0, 0)),
        out_shape=jax.ShapeDtypeStruct((b, D, 128), jnp.float32),
        compiler_params=pltpu.CompilerParams(
            dimension_semantics=("arbitrary", "arbitrary")),
    )(features, labels)

    task_ids = (2 * tasks[:, None]
                + jnp.arange(2, dtype=jnp.int32)[None, :]).reshape(TB)
    tidc = jnp.broadcast_to(task_ids[:, None], (TB, TB))
    tidr = jnp.broadcast_to(task_ids[None, :], (TB, TB))

    loss = pl.pallas_call(
        _stage2_kernel,
        out_shape=jax.ShapeDtypeStruct((1, 1), jnp.float32),
    )(sums, tidc, tidr, jnp.asarray(_sel))
    return loss[0, 0]


# speedup vs baseline: 1.4074x; 1.4074x over previous
import numpy as np
import jax
import jax.numpy as jnp
from jax.experimental import pallas as pl
from jax.experimental.pallas import tpu as pltpu

N_NEG = 24
TEMP = 0.07
B, D, H, W = 16, 32, 224, 224
TB = 2 * B
HW = H * W

_perms = np.stack([np.random.default_rng(1000 + i).permutation(TB)[:N_NEG]
                   for i in range(TB)])
_sel = np.zeros((TB, TB), np.float32)
for _i in range(TB):
    _sel[_i, _perms[_i]] = 1.0


DQ = D // 2


def _stage1_kernel(f0_ref, f1_ref, lab_ref, out_ref):
    l0 = lab_ref[0, 0]
    l1 = lab_ref[0, 1]
    for q, fref in ((0, f0_ref), (1, f1_ref)):
        f = fref[0]
        st0 = jnp.sum(f * l0[None, :, :], axis=(1, 2), keepdims=True)
        st1 = jnp.sum(f * l1[None, :, :], axis=(1, 2), keepdims=True)
        tot = jnp.sum(f, axis=(1, 2), keepdims=True)
        out_ref[0, q * DQ:(q + 1) * DQ, 0:1] = st0.reshape(DQ, 1)
        out_ref[0, q * DQ:(q + 1) * DQ, 1:2] = st1.reshape(DQ, 1)
        out_ref[0, q * DQ:(q + 1) * DQ, 2:3] = tot.reshape(DQ, 1)
    cnt0 = jnp.sum(l0, axis=(0, 1), keepdims=True)
    cnt1 = jnp.sum(l1, axis=(0, 1), keepdims=True)
    out_ref[0, 0:1, 3:4] = cnt0
    out_ref[0, 0:1, 4:5] = cnt1


def _stage2_kernel(sums_ref, tidc_ref, tidr_ref, sel_ref, out_ref):
    sel = sel_ref[...]

    st_t = jnp.concatenate([sums_ref[b, :, 0:2] for b in range(B)], axis=1)
    tot_t = jnp.concatenate(
        [sums_ref[b, :, 2:3] for b in range(B) for _ in range(2)], axis=1)
    cnt_r = jnp.concatenate([sums_ref[b, 0:1, 3:5] for b in range(B)], axis=1)

    def normalize(v, c):
        v = v / jnp.maximum(c, 1.0)
        n = jnp.sqrt(jnp.sum(v * v, axis=0, keepdims=True))
        return v / jnp.maximum(n, 1e-12)

    tgt = normalize(st_t, cnt_r)
    bgd = normalize(tot_t - st_t, float(HW) - cnt_r)

    dn = (((0,), (0,)), ((), ()))
    lt = jax.lax.dot_general(tgt, tgt, dn,
                             preferred_element_type=jnp.float32) / TEMP
    lb = jax.lax.dot_general(tgt, bgd, dn,
                             preferred_element_type=jnp.float32) / TEMP

    same = tidc_ref[...] == tidr_ref[...]
    df = jnp.where(same, 0.0, 1.0)
    sm = 1.0 - df

    rr = jax.lax.broadcasted_iota(jnp.int32, (TB, TB), 0)
    cc = jax.lax.broadcasted_iota(jnp.int32, (TB, TB), 1)
    upper = jnp.where(rr < cc, 1.0, 0.0)
    dn2 = (((1,), (0,)), ((), ()))
    rd = jax.lax.dot_general(df, upper, dn2, preferred_element_type=jnp.float32)
    rs = jax.lax.dot_general(sm, upper, dn2, preferred_element_type=jnp.float32)
    n_diff = jnp.sum(df, axis=1, keepdims=True)
    posn = jnp.where(same, n_diff + rs, rd)

    ins = jnp.zeros((TB, TB), jnp.float32)
    for p in range(TB):
        ins = ins + sel[:, p:p + 1] * jnp.where(posn == float(p), 1.0, 0.0)

    val = jnp.where(same, lb, lt)
    vmask = jnp.where(ins > 0.5, val, -1e30)
    nmax = jnp.max(vmask, axis=1, keepdims=True)
    sumexp = jnp.sum(ins * jnp.exp(val - nmax), axis=1, keepdims=True)

    cond = same & (cc != tidc_ref[...])
    firstc = jnp.min(jnp.where(cond, cc, TB * 2), axis=1, keepdims=True)
    firstc = jnp.where(firstc == TB * 2, 0, firstc)
    pos_logit = jnp.sum(lt * jnp.where(cc == firstc, 1.0, 0.0),
                        axis=1, keepdims=True)

    loss_i = jnp.log(sumexp) - (pos_logit - nmax)
    out_ref[...] = jnp.sum(loss_i, axis=0, keepdims=True) / TB


def kernel(features, labels, tasks):
    b = features.shape[0]

    sums = pl.pallas_call(
        _stage1_kernel,
        grid=(b,),
        in_specs=[
            pl.BlockSpec((1, DQ, H, W), lambda i: (i, 0, 0, 0)),
            pl.BlockSpec((1, DQ, H, W), lambda i: (i, 1, 0, 0)),
            pl.BlockSpec((1, 2, H, W), lambda i: (i, 0, 0, 0)),
        ],
        out_specs=pl.BlockSpec((1, D, 128), lambda i: (i, 0, 0)),
        out_shape=jax.ShapeDtypeStruct((b, D, 128), jnp.float32),
        compiler_params=pltpu.CompilerParams(
            dimension_semantics=("arbitrary",)),
    )(features, features, labels)

    task_ids = (2 * tasks[:, None]
                + jnp.arange(2, dtype=jnp.int32)[None, :]).reshape(TB)
    tidc = jnp.broadcast_to(task_ids[:, None], (TB, TB))
    tidr = jnp.broadcast_to(task_ids[None, :], (TB, TB))

    loss = pl.pallas_call(
        _stage2_kernel,
        out_shape=jax.ShapeDtypeStruct((1, 1), jnp.float32),
    )(sums, tidc, tidr, jnp.asarray(_sel))
    return loss[0, 0]


# speedup vs baseline: 1.4810x; 1.0523x over previous
import numpy as np
import jax
import jax.numpy as jnp
from jax.experimental import pallas as pl
from jax.experimental.pallas import tpu as pltpu

N_NEG = 24
TEMP = 0.07
B, D, H, W = 16, 32, 224, 224
TB = 2 * B
HW = H * W

_perms = np.stack([np.random.default_rng(1000 + i).permutation(TB)[:N_NEG]
                   for i in range(TB)])
_sel = np.zeros((TB, TB), np.float32)
for _i in range(TB):
    _sel[_i, _perms[_i]] = 1.0


def _finish_loss(acc_ref, tidc_ref, tidr_ref, sel_ref, out_ref):
    sel = sel_ref[...]

    st_t = jnp.concatenate([acc_ref[b, :, 0:2] for b in range(B)], axis=1)
    tot_t = jnp.concatenate(
        [acc_ref[b, :, 2:3] for b in range(B) for _ in range(2)], axis=1)
    cnt_r = jnp.concatenate([acc_ref[b, 0:1, 3:5] for b in range(B)], axis=1)

    def normalize(v, c):
        v = v / jnp.maximum(c, 1.0)
        n = jnp.sqrt(jnp.sum(v * v, axis=0, keepdims=True))
        return v / jnp.maximum(n, 1e-12)

    tgt = normalize(st_t, cnt_r)
    bgd = normalize(tot_t - st_t, float(HW) - cnt_r)

    dn = (((0,), (0,)), ((), ()))
    lt = jax.lax.dot_general(tgt, tgt, dn,
                             preferred_element_type=jnp.float32) / TEMP
    lb = jax.lax.dot_general(tgt, bgd, dn,
                             preferred_element_type=jnp.float32) / TEMP

    same = tidc_ref[...] == tidr_ref[...]
    df = jnp.where(same, 0.0, 1.0)
    sm = 1.0 - df

    rr = jax.lax.broadcasted_iota(jnp.int32, (TB, TB), 0)
    cc = jax.lax.broadcasted_iota(jnp.int32, (TB, TB), 1)
    upper = jnp.where(rr < cc, 1.0, 0.0)
    dn2 = (((1,), (0,)), ((), ()))
    rd = jax.lax.dot_general(df, upper, dn2, preferred_element_type=jnp.float32)
    rs = jax.lax.dot_general(sm, upper, dn2, preferred_element_type=jnp.float32)
    n_diff = jnp.sum(df, axis=1, keepdims=True)
    posn = jnp.where(same, n_diff + rs, rd)

    ins = jnp.zeros((TB, TB), jnp.float32)
    for p in range(TB):
        ins = ins + sel[:, p:p + 1] * jnp.where(posn == float(p), 1.0, 0.0)

    val = jnp.where(same, lb, lt)
    vmask = jnp.where(ins > 0.5, val, -1e30)
    nmax = jnp.max(vmask, axis=1, keepdims=True)
    sumexp = jnp.sum(ins * jnp.exp(val - nmax), axis=1, keepdims=True)

    cond = same & (cc != tidc_ref[...])
    firstc = jnp.min(jnp.where(cond, cc, TB * 2), axis=1, keepdims=True)
    firstc = jnp.where(firstc == TB * 2, 0, firstc)
    pos_logit = jnp.sum(lt * jnp.where(cc == firstc, 1.0, 0.0),
                        axis=1, keepdims=True)

    loss_i = jnp.log(sumexp) - (pos_logit - nmax)
    out_ref[...] = jnp.sum(loss_i, axis=0, keepdims=True) / TB


def _fused_kernel(feat_ref, lab_ref, tidc_ref, tidr_ref, sel_ref,
                  out_ref, acc_ref):
    i = pl.program_id(0)
    f = feat_ref[0]
    l0 = lab_ref[0, 0]
    l1 = lab_ref[0, 1]
    st0 = jnp.sum(f * l0[None, :, :], axis=(1, 2), keepdims=True)
    st1 = jnp.sum(f * l1[None, :, :], axis=(1, 2), keepdims=True)
    tot = jnp.sum(f, axis=(1, 2), keepdims=True)
    acc_ref[i, :, 0:1] = st0.reshape(D, 1)
    acc_ref[i, :, 1:2] = st1.reshape(D, 1)
    acc_ref[i, :, 2:3] = tot.reshape(D, 1)
    cnt0 = jnp.sum(l0, axis=(0, 1), keepdims=True)
    cnt1 = jnp.sum(l1, axis=(0, 1), keepdims=True)
    acc_ref[i, 0:1, 3:4] = cnt0
    acc_ref[i, 0:1, 4:5] = cnt1

    @pl.when(i == B - 1)
    def _finish():
        _finish_loss(acc_ref, tidc_ref, tidr_ref, sel_ref, out_ref)


def kernel(features, labels, tasks):
    b = features.shape[0]
    task_ids = (2 * tasks[:, None]
                + jnp.arange(2, dtype=jnp.int32)[None, :]).reshape(TB)
    tidc = jnp.broadcast_to(task_ids[:, None], (TB, TB))
    tidr = jnp.broadcast_to(task_ids[None, :], (TB, TB))

    loss = pl.pallas_call(
        _fused_kernel,
        grid=(b,),
        in_specs=[
            pl.BlockSpec((1, D, H, W), lambda i: (i, 0, 0, 0)),
            pl.BlockSpec((1, 2, H, W), lambda i: (i, 0, 0, 0)),
            pl.BlockSpec((TB, TB), lambda i: (0, 0)),
            pl.BlockSpec((TB, TB), lambda i: (0, 0)),
            pl.BlockSpec((TB, TB), lambda i: (0, 0)),
        ],
        out_specs=pl.BlockSpec((1, 1), lambda i: (0, 0)),
        out_shape=jax.ShapeDtypeStruct((1, 1), jnp.float32),
        scratch_shapes=[pltpu.VMEM((B, D, 128), jnp.float32)],
        compiler_params=pltpu.CompilerParams(
            dimension_semantics=("arbitrary",)),
    )(features, labels, tidc, tidr, jnp.asarray(_sel))
    return loss[0, 0]
